# baseline (speedup 1.0000x reference)
import functools

import jax
import jax.numpy as jnp
from jax import lax
from jax.experimental import pallas as pl
from jax.experimental.pallas import tpu as pltpu
from jax.experimental.pallas import tpu_sc as plsc

POS_THRESH = 0.6
COORD_SCALE = 1.0
GL = 32


def _dot(a, b):
    return lax.dot_general(
        a, b, (((1,), (0,)), ((), ())),
        precision=lax.Precision.HIGHEST,
        preferred_element_type=jnp.float32)


def _body(truth_ref, trv_ref, biases_ref, xy_ref, wh_ref, obj_ref,
          txy_ref, twh_ref, tobj_ref, tnoobj_ref):
    H, W = xy_ref.shape[2], xy_ref.shape[3]
    A = xy_ref.shape[1] // 2
    T = truth_ref.shape[2] // 5

    row_i = lax.broadcasted_iota(jnp.int32, (H, W), 0)
    col_i = lax.broadcasted_iota(jnp.int32, (H, W), 1)
    ii = col_i.astype(jnp.float32)
    jj = row_i.astype(jnp.float32)

    tx = trv_ref[0, 0:1, :]
    ty = trv_ref[0, 1:2, :]
    tw = trv_ref[0, 2:3, :]
    th = trv_ref[0, 3:4, :]

    ti = (tx * W).astype(jnp.int32)
    tj = (ty * H).astype(jnp.int32)
    ti = jnp.where(ti >= W, W, ti)
    tj = jnp.where(tj >= H, H, tj)
    inval = (tx <= 0) | (tx >= 1) | (ty <= 0) | (ty >= 1)
    ti = jnp.where(inval, -1, ti)
    tj = jnp.where(inval, -1, tj)

    best = jnp.full((1, GL), -jnp.inf, jnp.float32)
    nvec = jnp.zeros((1, GL), jnp.int32)
    for a in range(A):
        b0 = biases_ref[a, 0]
        b1 = biases_ref[a, 1]
        il2 = jnp.maximum(-b0 / 2 / W, -tw / 2)
        ir2 = jnp.minimum(b0 / 2 / W, tw / 2)
        it2 = jnp.maximum(-b1 / 2 / H, -th / 2)
        ib2 = jnp.minimum(b1 / 2 / H, th / 2)
        ov2 = jnp.maximum(ir2 - il2, 0.0) * jnp.maximum(ib2 - it2, 0.0)
        iou2 = ov2 / (b0 * b1 / W / H + tw * th - ov2)
        take = iou2 > best
        nvec = jnp.where(take, a, nvec)
        best = jnp.where(take, iou2, best)
    tn = jnp.where(inval, -1, nvec)

    valid = (ti >= 0) & (tj >= 0) & (tj < H) & (ti < W) & (tw > 0) & (th > 0)
    icv = jnp.clip(ti, 0, W - 1)
    jcv = jnp.clip(tj, 0, H - 1)
    ncv = jnp.clip(tn, 0, A - 1)
    fiv = icv.astype(jnp.float32)
    fjv = jcv.astype(jnp.float32)

    keyf = ((ncv * H + jcv) * W + icv + 1).astype(jnp.float32)
    key_l = jnp.where(valid, keyf, -2.0)
    key_for_t = jnp.where(valid, keyf, -1.0)
    sub_l = lax.broadcasted_iota(jnp.int32, (H, GL), 1)
    diag = (lax.broadcasted_iota(jnp.int32, (H, GL), 0) == sub_l)
    ones_col = jnp.ones((GL, 1), jnp.float32)

    def transpose_lanes(v):
        m = jnp.where(diag, jnp.broadcast_to(v, (H, GL)), 0.0)
        return _dot(m, ones_col)

    key_s = transpose_lanes(key_for_t)
    g_sub = lax.broadcasted_iota(jnp.int32, (H, GL), 0)
    killer = (jnp.broadcast_to(key_s, (H, GL)) == jnp.broadcast_to(key_l, (H, GL))) \
        & (g_sub > sub_l)
    killed = jnp.sum(jnp.where(killer, 1.0, 0.0), axis=0, keepdims=True) > 0.5
    alive = valid & (~killed)

    rowm = jnp.where(
        lax.broadcasted_iota(jnp.int32, (H, GL), 0)
        == jnp.broadcast_to(jcv, (H, GL)), 1.0, 0.0)
    colm = jnp.where(
        lax.broadcasted_iota(jnp.int32, (W, GL), 0)
        == jnp.broadcast_to(icv, (W, GL)), 1.0, 0.0)
    ic_s = transpose_lanes(fiv)
    colmT = jnp.where(
        jnp.broadcast_to(ic_s, (H, W))
        == lax.broadcasted_iota(jnp.int32, (H, W), 1).astype(jnp.float32),
        1.0, 0.0)[0:GL, :]

    s_x = jnp.zeros((1, GL), jnp.float32)
    s_y = jnp.zeros((1, GL), jnp.float32)
    s_bw = jnp.zeros((1, GL), jnp.float32)
    s_bh = jnp.zeros((1, GL), jnp.float32)
    zero = jnp.zeros((H, W), jnp.float32)
    for a in range(A):
        x = xy_ref[0, a]
        y = xy_ref[0, a + A]
        w = wh_ref[0, a]
        h = wh_ref[0, a + A]
        bx = (x + ii) / W
        by = (y + jj) / H
        bw = jnp.exp(w) * biases_ref[a, 0] / W
        bh = jnp.exp(h) * biases_ref[a, 1] / H
        a1 = bw * bh
        bxl = bx - bw / 2
        bxr = bx + bw / 2
        byt = by - bh / 2
        byb = by + bh / 2

        UNROLL = 3 if T % 3 == 0 else (2 if T % 2 == 0 else 1)

        def one_g(g, miou):
            gtx = truth_ref[0, 0, 5 * g]
            gty = truth_ref[0, 0, 5 * g + 1]
            gtw = truth_ref[0, 0, 5 * g + 2]
            gth = truth_ref[0, 0, 5 * g + 3]
            il = jnp.maximum(bxl, gtx - gtw / 2)
            ir = jnp.minimum(bxr, gtx + gtw / 2)
            it = jnp.maximum(byt, gty - gth / 2)
            ib = jnp.minimum(byb, gty + gth / 2)
            ov = jnp.maximum(ir - il, 0.0) * jnp.maximum(ib - it, 0.0)
            iou = ov / (a1 + gtw * gth - ov)
            return jnp.maximum(miou, iou)

        def g_body(k, miou):
            g = k * UNROLL
            for u in range(UNROLL):
                miou = one_g(g + u, miou)
            return miou

        miou = lax.fori_loop(0, T // UNROLL, g_body, zero)
        tnoobj_ref[0, a] = jnp.where(miou > POS_THRESH, obj_ref[0, a], 0.0)

        stack = jnp.concatenate([x, y, bw, bh], axis=0)
        m1 = _dot(stack, colm)
        in_a = ncv == a
        s_x = jnp.where(in_a, jnp.sum(rowm * m1[0:H], 0, keepdims=True), s_x)
        s_y = jnp.where(in_a, jnp.sum(rowm * m1[H:2 * H], 0, keepdims=True), s_y)
        s_bw = jnp.where(in_a, jnp.sum(rowm * m1[2 * H:3 * H], 0, keepdims=True), s_bw)
        s_bh = jnp.where(in_a, jnp.sum(rowm * m1[3 * H:4 * H], 0, keepdims=True), s_bh)

    b0n = jnp.zeros((1, GL), jnp.float32)
    b1n = jnp.zeros((1, GL), jnp.float32)
    for a in range(A):
        in_a = ncv == a
        b0n = jnp.where(in_a, biases_ref[a, 0], b0n)
        b1n = jnp.where(in_a, biases_ref[a, 1], b1n)
    v_x = tx * W - fiv
    v_y = ty * H - fjv
    safe_tw = jnp.where(alive, tw, 1.0)
    safe_th = jnp.where(alive, th, 1.0)
    v_w = jnp.log(safe_tw * W / jnp.where(alive, b0n, 1.0))
    v_h = jnp.log(safe_th * H / jnp.where(alive, b1n, 1.0))

    bx_s = (s_x + fiv) / W
    by_s = (s_y + fjv) / H
    il = jnp.maximum(bx_s - s_bw / 2, tx - tw / 2)
    ir = jnp.minimum(bx_s + s_bw / 2, tx + tw / 2)
    it = jnp.maximum(by_s - s_bh / 2, ty - th / 2)
    ib = jnp.minimum(by_s + s_bh / 2, ty + th / 2)
    ov = jnp.maximum(ir - il, 0.0) * jnp.maximum(ib - it, 0.0)
    den = s_bw * s_bh + tw * th - ov
    iou_s = ov / jnp.where(alive, den, 1.0)

    alive_f = jnp.where(alive, 1.0, 0.0)
    vals = [jnp.where(alive, v, 0.0)
            for v in (v_x, v_y, v_w, v_h, iou_s)]
    vals.append(alive_f)

    for a in range(A):
        sel = jnp.where(ncv == a, 1.0, 0.0) * alive_f
        astack = jnp.concatenate(
            [rowm * jnp.broadcast_to(sel * v, (H, GL)) for v in vals], axis=0)
        planes = _dot(astack, colmT)
        Vx, Vy, Vw, Vh, Viou, Vhit = (
            planes[k * H:(k + 1) * H] for k in range(6))
        hit = Vhit > 0.5
        txy_ref[0, a] = jnp.where(hit, Vx, xy_ref[0, a])
        txy_ref[0, a + A] = jnp.where(hit, Vy, xy_ref[0, a + A])
        twh_ref[0, a] = jnp.where(hit, Vw, wh_ref[0, a])
        twh_ref[0, a + A] = jnp.where(hit, Vh, wh_ref[0, a + A])
        o = obj_ref[0, a]
        tobj_ref[0, a] = jnp.where(hit, Viou, o)
        tnoobj_ref[0, a] = jnp.where(hit, o, tnoobj_ref[0, a])


def _build(B, A, H, W, T, interpret=False):
    A2 = 2 * A
    big = lambda c: pl.BlockSpec((1, c, H, W), lambda b: (b, 0, 0, 0))
    in_specs = [
        pl.BlockSpec((1, 1, 5 * T), lambda b: (b, 0, 0), memory_space=pltpu.SMEM),
        pl.BlockSpec((1, 8, GL), lambda b: (b, 0, 0)),
        pl.BlockSpec((A, 2), lambda b: (0, 0), memory_space=pltpu.SMEM),
        big(A2), big(A2), big(A),
    ]
    out_specs = [big(A2), big(A2), big(A), big(A)]
    shp = lambda c: jax.ShapeDtypeStruct((B, c, H, W), jnp.float32)
    out_shape = [shp(A2), shp(A2), shp(A), shp(A)]
    return pl.pallas_call(
        _body,
        grid=(B,),
        in_specs=in_specs,
        out_specs=out_specs,
        out_shape=out_shape,
        compiler_params=pltpu.CompilerParams(
            dimension_semantics=("arbitrary",)),
        interpret=interpret,
    )


def _sc_build(B, A, H, W, T):
    A2 = 2 * A
    mesh = plsc.VectorSubcoreMesh(core_axis_name="c", subcore_axis_name="s")

    @functools.partial(
        pl.kernel, mesh=mesh,
        out_type=[jax.ShapeDtypeStruct((B, A2 * H * W), jnp.float32),
                  jax.ShapeDtypeStruct((B, A * H * W), jnp.float32)],
        scratch_types=[
            pltpu.VMEM((8 * GL,), jnp.float32),
            pltpu.VMEM((2 * A * 16,), jnp.float32),
            pltpu.VMEM((A * H * W,), jnp.float32),
        ],
    )
    def sc_kern(trv_hbm, biasb_hbm, tww_hbm, tlabel_hbm,
                trv_v, bias_v, slab):
        wid = lax.axis_index("s") * 2 + lax.axis_index("c")
        b = wid // 3
        kind = wid - 3 * b

        @pl.when(wid < 3 * B)
        def _():
            pltpu.sync_copy(trv_hbm.at[b], trv_v)
            pltpu.sync_copy(biasb_hbm, bias_v)

            lane = lax.broadcasted_iota(jnp.int32, (16,), 0)
            ncf, jcf, icf, vlf, okf = [], [], [], [], []
            for hchunk in range(2):
                off = 16 * hchunk
                tx = trv_v[pl.ds(0 * GL + off, 16)]
                ty = trv_v[pl.ds(1 * GL + off, 16)]
                tw = trv_v[pl.ds(2 * GL + off, 16)]
                th = trv_v[pl.ds(3 * GL + off, 16)]
                cl = trv_v[pl.ds(4 * GL + off, 16)]

                ti = (tx * W).astype(jnp.int32)
                tj = (ty * H).astype(jnp.int32)
                ti = jnp.where(ti >= W, W, ti)
                tj = jnp.where(tj >= H, H, tj)
                inval = (tx <= 0.0) | (tx >= 1.0) | (ty <= 0.0) | (ty >= 1.0)
                ti = jnp.where(inval, -1, ti)
                tj = jnp.where(inval, -1, tj)

                best = jnp.full((16,), -jnp.inf, jnp.float32)
                nvec = jnp.zeros((16,), jnp.int32)
                for a in range(A):
                    b0 = bias_v[pl.ds(32 * a, 16)]
                    b1 = bias_v[pl.ds(32 * a + 16, 16)]
                    il2 = jnp.maximum(-b0 / 2 / W, -tw / 2)
                    ir2 = jnp.minimum(b0 / 2 / W, tw / 2)
                    it2 = jnp.maximum(-b1 / 2 / H, -th / 2)
                    ib2 = jnp.minimum(b1 / 2 / H, th / 2)
                    ov2 = jnp.maximum(ir2 - il2, 0.0) * jnp.maximum(ib2 - it2, 0.0)
                    iou2 = ov2 / (b0 * b1 / W / H + tw * th - ov2)
                    take = iou2 > best
                    nvec = jnp.where(take, a, nvec)
                    best = jnp.where(take, iou2, best)
                tn = jnp.where(inval, -1, nvec)

                valid = (ti >= 0) & (tj >= 0) & (tj < H) & (ti < W) \
                    & (tw > 0) & (th > 0)
                icv = jnp.clip(ti, 0, W - 1)
                jcv = jnp.clip(tj, 0, H - 1)
                ncv = jnp.clip(tn, 0, A - 1)
                wgt = COORD_SCALE * (2.0 - tw * th)

                ncf.append(ncv.astype(jnp.float32))
                jcf.append(jcv.astype(jnp.float32))
                icf.append(icv.astype(jnp.float32))
                vlf.append(jnp.where(kind == 2, cl, wgt))
                okf.append(jnp.where(valid, 1.0, 0.0))

            init = jnp.where(kind == 2, jnp.full((16,), -1.0, jnp.float32),
                             jnp.zeros((16,), jnp.float32))
            def ms(i, carry):
                slab[pl.ds(i * 16, 16)] = init
                return carry

            lax.fori_loop(0, A * H * W // 16, ms, 0)

            def pg(g, carry):
                gl16 = g % 16
                ch = g // 16

                def ext(vecs):
                    return vecs[ch][gl16]

                ncs = ext(ncf).astype(jnp.int32)
                jcs = ext(jcf).astype(jnp.int32)
                ics = ext(icf).astype(jnp.int32)
                vls = ext(vlf)
                oks = ext(okf)
                flat = (ncs * H + jcs) * W + ics
                w0 = pl.multiple_of((flat // 16) * 16, 16)
                li = flat - w0

                @pl.when(oks > 0.5)
                def _():
                    win = slab[pl.ds(w0, 16)]
                    slab[pl.ds(w0, 16)] = jnp.where(
                        lane == li, jnp.full((16,), vls), win)
                return carry

            for g in range(T):
                pg(g, 0)

            @pl.when(kind == 0)
            def _():
                pltpu.sync_copy(slab, tww_hbm.at[b, pl.ds(0, A * H * W)])

            @pl.when(kind == 1)
            def _():
                pltpu.sync_copy(slab, tww_hbm.at[b, pl.ds(A * H * W, A * H * W)])

            @pl.when(kind == 2)
            def _():
                pltpu.sync_copy(slab, tlabel_hbm.at[b])

    return sc_kern


def kernel(xy, wh, obj, truth, biases):
    xy = lax.stop_gradient(xy)
    wh = lax.stop_gradient(wh)
    obj = lax.stop_gradient(obj)
    B, A2, H, W = xy.shape
    A = A2 // 2
    T = truth.shape[1] // 5
    truth5 = truth.reshape(B, T, 5).transpose(0, 2, 1)
    trv = jnp.zeros((B, 8, GL), jnp.float32).at[:, :5, :T].set(truth5)
    biasb = jnp.repeat(biases.reshape(-1)[:, None], 16, axis=1)

    call = _build(B, A, H, W, T)
    t_xy, t_wh, t_obj, t_noobj = call(
        truth.reshape(B, 1, 5 * T), trv, biases, xy, wh, obj)
    sc = _sc_build(B, A, H, W, T)
    t_ww, t_label = sc(trv.reshape(B, 8 * GL), biasb.reshape(-1))
    t_ww = t_ww.reshape(B, A2, H, W)
    t_label = t_label.reshape(B, A, H, W)
    return (t_xy, t_wh, t_ww, t_obj, t_noobj, t_label)


# speedup vs baseline: 1.0001x; 1.0001x over previous
import functools

import jax
import jax.numpy as jnp
from jax import lax
from jax.experimental import pallas as pl
from jax.experimental.pallas import tpu as pltpu
from jax.experimental.pallas import tpu_sc as plsc

POS_THRESH = 0.6
COORD_SCALE = 1.0
GL = 32


def _dot(a, b):
    return lax.dot_general(
        a, b, (((1,), (0,)), ((), ())),
        precision=lax.Precision.HIGHEST,
        preferred_element_type=jnp.float32)


def _body(truth_ref, trv_ref, biases_ref, xy_ref, wh_ref, obj_ref,
          txy_ref, twh_ref, tobj_ref, tnoobj_ref):
    H, W = xy_ref.shape[2], xy_ref.shape[3]
    A = xy_ref.shape[1] // 2
    T = truth_ref.shape[2] // 5

    row_i = lax.broadcasted_iota(jnp.int32, (H, W), 0)
    col_i = lax.broadcasted_iota(jnp.int32, (H, W), 1)
    ii = col_i.astype(jnp.float32)
    jj = row_i.astype(jnp.float32)

    tx = trv_ref[0, 0:1, :]
    ty = trv_ref[0, 1:2, :]
    tw = trv_ref[0, 2:3, :]
    th = trv_ref[0, 3:4, :]

    ti = (tx * W).astype(jnp.int32)
    tj = (ty * H).astype(jnp.int32)
    ti = jnp.where(ti >= W, W, ti)
    tj = jnp.where(tj >= H, H, tj)
    inval = (tx <= 0) | (tx >= 1) | (ty <= 0) | (ty >= 1)
    ti = jnp.where(inval, -1, ti)
    tj = jnp.where(inval, -1, tj)

    best = jnp.full((1, GL), -jnp.inf, jnp.float32)
    nvec = jnp.zeros((1, GL), jnp.int32)
    for a in range(A):
        b0 = biases_ref[a, 0]
        b1 = biases_ref[a, 1]
        il2 = jnp.maximum(-b0 / 2 / W, -tw / 2)
        ir2 = jnp.minimum(b0 / 2 / W, tw / 2)
        it2 = jnp.maximum(-b1 / 2 / H, -th / 2)
        ib2 = jnp.minimum(b1 / 2 / H, th / 2)
        ov2 = jnp.maximum(ir2 - il2, 0.0) * jnp.maximum(ib2 - it2, 0.0)
        iou2 = ov2 / (b0 * b1 / W / H + tw * th - ov2)
        take = iou2 > best
        nvec = jnp.where(take, a, nvec)
        best = jnp.where(take, iou2, best)
    tn = jnp.where(inval, -1, nvec)

    valid = (ti >= 0) & (tj >= 0) & (tj < H) & (ti < W) & (tw > 0) & (th > 0)
    icv = jnp.clip(ti, 0, W - 1)
    jcv = jnp.clip(tj, 0, H - 1)
    ncv = jnp.clip(tn, 0, A - 1)
    fiv = icv.astype(jnp.float32)
    fjv = jcv.astype(jnp.float32)

    keyf = ((ncv * H + jcv) * W + icv + 1).astype(jnp.float32)
    key_l = jnp.where(valid, keyf, -2.0)
    key_for_t = jnp.where(valid, keyf, -1.0)
    sub_l = lax.broadcasted_iota(jnp.int32, (H, GL), 1)
    diag = (lax.broadcasted_iota(jnp.int32, (H, GL), 0) == sub_l)
    ones_col = jnp.ones((GL, 1), jnp.float32)

    def transpose_lanes(v):
        m = jnp.where(diag, jnp.broadcast_to(v, (H, GL)), 0.0)
        return _dot(m, ones_col)

    key_s = transpose_lanes(key_for_t)
    g_sub = lax.broadcasted_iota(jnp.int32, (H, GL), 0)
    killer = (jnp.broadcast_to(key_s, (H, GL)) == jnp.broadcast_to(key_l, (H, GL))) \
        & (g_sub > sub_l)
    killed = jnp.sum(jnp.where(killer, 1.0, 0.0), axis=0, keepdims=True) > 0.5
    alive = valid & (~killed)

    rowm = jnp.where(
        lax.broadcasted_iota(jnp.int32, (H, GL), 0)
        == jnp.broadcast_to(jcv, (H, GL)), 1.0, 0.0)
    colm = jnp.where(
        lax.broadcasted_iota(jnp.int32, (W, GL), 0)
        == jnp.broadcast_to(icv, (W, GL)), 1.0, 0.0)
    ic_s = transpose_lanes(fiv)
    colmT = jnp.where(
        jnp.broadcast_to(ic_s, (H, W))
        == lax.broadcasted_iota(jnp.int32, (H, W), 1).astype(jnp.float32),
        1.0, 0.0)[0:GL, :]

    s_x = jnp.zeros((1, GL), jnp.float32)
    s_y = jnp.zeros((1, GL), jnp.float32)
    s_bw = jnp.zeros((1, GL), jnp.float32)
    s_bh = jnp.zeros((1, GL), jnp.float32)
    zero = jnp.zeros((H, W), jnp.float32)
    for a in range(A):
        x = xy_ref[0, a]
        y = xy_ref[0, a + A]
        w = wh_ref[0, a]
        h = wh_ref[0, a + A]
        bx = (x + ii) / W
        by = (y + jj) / H
        bw = jnp.exp(w) * biases_ref[a, 0] / W
        bh = jnp.exp(h) * biases_ref[a, 1] / H
        a1 = bw * bh
        bxl = bx - bw / 2
        bxr = bx + bw / 2
        byt = by - bh / 2
        byb = by + bh / 2

        UNROLL = 3 if T % 3 == 0 else (2 if T % 2 == 0 else 1)

        def one_g(g, miou):
            gtx = truth_ref[0, 0, 5 * g]
            gty = truth_ref[0, 0, 5 * g + 1]
            gtw = truth_ref[0, 0, 5 * g + 2]
            gth = truth_ref[0, 0, 5 * g + 3]
            il = jnp.maximum(bxl, gtx - gtw / 2)
            ir = jnp.minimum(bxr, gtx + gtw / 2)
            it = jnp.maximum(byt, gty - gth / 2)
            ib = jnp.minimum(byb, gty + gth / 2)
            ov = jnp.maximum(ir - il, 0.0) * jnp.maximum(ib - it, 0.0)
            iou = ov / (a1 + gtw * gth - ov)
            return jnp.maximum(miou, iou)

        def g_body(k, miou):
            g = k * UNROLL
            for u in range(UNROLL):
                miou = one_g(g + u, miou)
            return miou

        miou = lax.fori_loop(0, T // UNROLL, g_body, zero)
        tnoobj_ref[0, a] = jnp.where(miou > POS_THRESH, obj_ref[0, a], 0.0)

        stack = jnp.concatenate([x, y, bw, bh], axis=0)
        m1 = _dot(stack, colm)
        in_a = ncv == a
        s_x = jnp.where(in_a, jnp.sum(rowm * m1[0:H], 0, keepdims=True), s_x)
        s_y = jnp.where(in_a, jnp.sum(rowm * m1[H:2 * H], 0, keepdims=True), s_y)
        s_bw = jnp.where(in_a, jnp.sum(rowm * m1[2 * H:3 * H], 0, keepdims=True), s_bw)
        s_bh = jnp.where(in_a, jnp.sum(rowm * m1[3 * H:4 * H], 0, keepdims=True), s_bh)

    b0n = jnp.zeros((1, GL), jnp.float32)
    b1n = jnp.zeros((1, GL), jnp.float32)
    for a in range(A):
        in_a = ncv == a
        b0n = jnp.where(in_a, biases_ref[a, 0], b0n)
        b1n = jnp.where(in_a, biases_ref[a, 1], b1n)
    v_x = tx * W - fiv
    v_y = ty * H - fjv
    safe_tw = jnp.where(alive, tw, 1.0)
    safe_th = jnp.where(alive, th, 1.0)
    v_w = jnp.log(safe_tw * W / jnp.where(alive, b0n, 1.0))
    v_h = jnp.log(safe_th * H / jnp.where(alive, b1n, 1.0))

    bx_s = (s_x + fiv) / W
    by_s = (s_y + fjv) / H
    il = jnp.maximum(bx_s - s_bw / 2, tx - tw / 2)
    ir = jnp.minimum(bx_s + s_bw / 2, tx + tw / 2)
    it = jnp.maximum(by_s - s_bh / 2, ty - th / 2)
    ib = jnp.minimum(by_s + s_bh / 2, ty + th / 2)
    ov = jnp.maximum(ir - il, 0.0) * jnp.maximum(ib - it, 0.0)
    den = s_bw * s_bh + tw * th - ov
    iou_s = ov / jnp.where(alive, den, 1.0)

    alive_f = jnp.where(alive, 1.0, 0.0)
    vals = [jnp.where(alive, v, 0.0)
            for v in (v_x, v_y, v_w, v_h, iou_s)]
    vals.append(alive_f)

    for a in range(A):
        sel = jnp.where(ncv == a, 1.0, 0.0) * alive_f
        astack = jnp.concatenate(
            [rowm * jnp.broadcast_to(sel * v, (H, GL)) for v in vals], axis=0)
        planes = _dot(astack, colmT)
        Vx, Vy, Vw, Vh, Viou, Vhit = (
            planes[k * H:(k + 1) * H] for k in range(6))
        hit = Vhit > 0.5
        txy_ref[0, a] = jnp.where(hit, Vx, xy_ref[0, a])
        txy_ref[0, a + A] = jnp.where(hit, Vy, xy_ref[0, a + A])
        twh_ref[0, a] = jnp.where(hit, Vw, wh_ref[0, a])
        twh_ref[0, a + A] = jnp.where(hit, Vh, wh_ref[0, a + A])
        o = obj_ref[0, a]
        tobj_ref[0, a] = jnp.where(hit, Viou, o)
        tnoobj_ref[0, a] = jnp.where(hit, o, tnoobj_ref[0, a])


def _build(B, A, H, W, T, interpret=False):
    A2 = 2 * A
    big = lambda c: pl.BlockSpec((1, c, H, W), lambda b: (b, 0, 0, 0))
    in_specs = [
        pl.BlockSpec((1, 1, 5 * T), lambda b: (b, 0, 0), memory_space=pltpu.SMEM),
        pl.BlockSpec((1, 8, GL), lambda b: (b, 0, 0)),
        pl.BlockSpec((A, 2), lambda b: (0, 0), memory_space=pltpu.SMEM),
        big(A2), big(A2), big(A),
    ]
    out_specs = [big(A2), big(A2), big(A), big(A)]
    shp = lambda c: jax.ShapeDtypeStruct((B, c, H, W), jnp.float32)
    out_shape = [shp(A2), shp(A2), shp(A), shp(A)]
    return pl.pallas_call(
        _body,
        grid=(B,),
        in_specs=in_specs,
        out_specs=out_specs,
        out_shape=out_shape,
        compiler_params=pltpu.CompilerParams(
            dimension_semantics=("arbitrary",)),
        interpret=interpret,
    )


def _sc_build(B, A, H, W, T):
    A2 = 2 * A
    mesh = plsc.VectorSubcoreMesh(core_axis_name="c", subcore_axis_name="s")

    @functools.partial(
        pl.kernel, mesh=mesh,
        out_type=[jax.ShapeDtypeStruct((B, A2 * H * W), jnp.float32),
                  jax.ShapeDtypeStruct((B, A * H * W), jnp.float32)],
        scratch_types=[
            pltpu.VMEM((8 * GL,), jnp.float32),
            pltpu.VMEM((2 * A * 16,), jnp.float32),
            pltpu.VMEM((A * H * W,), jnp.float32),
        ],
    )
    def sc_kern(trv_hbm, biasb_hbm, tww_hbm, tlabel_hbm,
                trv_v, bias_v, slab):
        wid = lax.axis_index("s") * 2 + lax.axis_index("c")
        b = wid // 3
        kind = wid - 3 * b

        @pl.when(wid < 3 * B)
        def _():
            pltpu.sync_copy(trv_hbm.at[b], trv_v)
            pltpu.sync_copy(biasb_hbm, bias_v)

            lane = lax.broadcasted_iota(jnp.int32, (16,), 0)
            ncf, jcf, icf, vlf, okf = [], [], [], [], []
            for hchunk in range(2):
                off = 16 * hchunk
                tx = trv_v[pl.ds(0 * GL + off, 16)]
                ty = trv_v[pl.ds(1 * GL + off, 16)]
                tw = trv_v[pl.ds(2 * GL + off, 16)]
                th = trv_v[pl.ds(3 * GL + off, 16)]
                cl = trv_v[pl.ds(4 * GL + off, 16)]

                ti = (tx * W).astype(jnp.int32)
                tj = (ty * H).astype(jnp.int32)
                ti = jnp.where(ti >= W, W, ti)
                tj = jnp.where(tj >= H, H, tj)
                inval = (tx <= 0.0) | (tx >= 1.0) | (ty <= 0.0) | (ty >= 1.0)
                ti = jnp.where(inval, -1, ti)
                tj = jnp.where(inval, -1, tj)

                best = jnp.full((16,), -jnp.inf, jnp.float32)
                nvec = jnp.zeros((16,), jnp.int32)
                for a in range(A):
                    b0 = bias_v[pl.ds(32 * a, 16)]
                    b1 = bias_v[pl.ds(32 * a + 16, 16)]
                    il2 = jnp.maximum(-b0 / 2 / W, -tw / 2)
                    ir2 = jnp.minimum(b0 / 2 / W, tw / 2)
                    it2 = jnp.maximum(-b1 / 2 / H, -th / 2)
                    ib2 = jnp.minimum(b1 / 2 / H, th / 2)
                    ov2 = jnp.maximum(ir2 - il2, 0.0) * jnp.maximum(ib2 - it2, 0.0)
                    iou2 = ov2 / (b0 * b1 / W / H + tw * th - ov2)
                    take = iou2 > best
                    nvec = jnp.where(take, a, nvec)
                    best = jnp.where(take, iou2, best)
                tn = jnp.where(inval, -1, nvec)

                valid = (ti >= 0) & (tj >= 0) & (tj < H) & (ti < W) \
                    & (tw > 0) & (th > 0)
                icv = jnp.clip(ti, 0, W - 1)
                jcv = jnp.clip(tj, 0, H - 1)
                ncv = jnp.clip(tn, 0, A - 1)
                wgt = COORD_SCALE * (2.0 - tw * th)

                ncf.append(ncv.astype(jnp.float32))
                jcf.append(jcv.astype(jnp.float32))
                icf.append(icv.astype(jnp.float32))
                vlf.append(jnp.where(kind == 2, cl, wgt))
                okf.append(jnp.where(valid, 1.0, 0.0))

            init = jnp.where(kind == 2, jnp.full((16,), -1.0, jnp.float32),
                             jnp.zeros((16,), jnp.float32))
            def ms(i, carry):
                for u in range(8):
                    slab[pl.ds((i * 8 + u) * 16, 16)] = init
                return carry

            lax.fori_loop(0, A * H * W // 128, ms, 0)

            def pg(g, carry):
                gl16 = g % 16
                ch = g // 16

                def ext(vecs):
                    return vecs[ch][gl16]

                ncs = ext(ncf).astype(jnp.int32)
                jcs = ext(jcf).astype(jnp.int32)
                ics = ext(icf).astype(jnp.int32)
                vls = ext(vlf)
                oks = ext(okf)
                flat = (ncs * H + jcs) * W + ics
                w0 = pl.multiple_of((flat // 16) * 16, 16)
                li = flat - w0

                @pl.when(oks > 0.5)
                def _():
                    win = slab[pl.ds(w0, 16)]
                    slab[pl.ds(w0, 16)] = jnp.where(
                        lane == li, jnp.full((16,), vls), win)
                return carry

            for g in range(T):
                pg(g, 0)

            @pl.when(kind == 0)
            def _():
                pltpu.sync_copy(slab, tww_hbm.at[b, pl.ds(0, A * H * W)])

            @pl.when(kind == 1)
            def _():
                pltpu.sync_copy(slab, tww_hbm.at[b, pl.ds(A * H * W, A * H * W)])

            @pl.when(kind == 2)
            def _():
                pltpu.sync_copy(slab, tlabel_hbm.at[b])

    return sc_kern


def kernel(xy, wh, obj, truth, biases):
    xy = lax.stop_gradient(xy)
    wh = lax.stop_gradient(wh)
    obj = lax.stop_gradient(obj)
    B, A2, H, W = xy.shape
    A = A2 // 2
    T = truth.shape[1] // 5
    truth5 = truth.reshape(B, T, 5).transpose(0, 2, 1)
    trv = jnp.zeros((B, 8, GL), jnp.float32).at[:, :5, :T].set(truth5)
    biasb = jnp.repeat(biases.reshape(-1)[:, None], 16, axis=1)

    sc = _sc_build(B, A, H, W, T)
    t_ww, t_label = sc(trv.reshape(B, 8 * GL), biasb.reshape(-1))
    call = _build(B, A, H, W, T)
    t_xy, t_wh, t_obj, t_noobj = call(
        truth.reshape(B, 1, 5 * T), trv, biases, xy, wh, obj)
    t_ww = t_ww.reshape(B, A2, H, W)
    t_label = t_label.reshape(B, A, H, W)
    return (t_xy, t_wh, t_ww, t_obj, t_noobj, t_label)
